# baseline (speedup 1.0000x reference)
import functools

import jax
import jax.numpy as jnp
import numpy as np
from jax import lax
from jax.experimental import pallas as pl
from jax.experimental.pallas import tpu as pltpu
from jax.experimental.pallas import tpu_sc as plsc

_DROP_PROB = 0.1
_L = 16


@functools.lru_cache(maxsize=None)
def _keep_const(B, T):
    with jax.ensure_compile_time_eval():
        u = jax.random.uniform(jax.random.key(42), (B, T))
        return np.asarray(u > _DROP_PROB).astype(np.int32)


@functools.lru_cache(maxsize=None)
def _sc_event_dropout(B, T, F):
    BT = B * T
    C = 32
    HALF = T // 2
    NCH = HALF // C
    NV_T = T // _L
    NV_F = F // _L

    mesh = plsc.VectorSubcoreMesh(core_axis_name="c", subcore_axis_name="s")

    @functools.partial(
        pl.kernel,
        out_type=(
            jax.ShapeDtypeStruct((BT, F), jnp.float32),
            jax.ShapeDtypeStruct((B, _L), jnp.int32),
        ),
        mesh=mesh,
        compiler_params=pltpu.CompilerParams(needs_layout_passes=False),
        scratch_types=[
            pltpu.VMEM((T + _L,), jnp.int32),
            pltpu.VMEM((T,), jnp.int32),
            pltpu.VMEM((_L,), jnp.int32),
            pltpu.VMEM((_L,), jnp.int32),
            pltpu.VMEM((C,), jnp.int32),
            pltpu.VMEM((C,), jnp.int32),
            pltpu.VMEM((C, F), jnp.float32),
            pltpu.VMEM((C, F), jnp.float32),
            pltpu.VMEM((C, F), jnp.float32),
            pltpu.SemaphoreType.DMA,
            pltpu.SemaphoreType.DMA,
            pltpu.SemaphoreType.DMA,
            pltpu.SemaphoreType.DMA,
        ],
    )
    def k(tensor_hbm, keep_hbm, lenin_hbm, out_hbm, len_hbm,
          idx_v, keep_v, len_v, lenin_v, cidx0, cidx1, gbuf0, gbuf1, zbuf,
          gsem0, gsem1, osem0, osem1):
        cidx = (cidx0, cidx1)
        gbufs = (gbuf0, gbuf1)
        gsem = (gsem0, gsem1)
        osem = (osem0, osem1)
        wid = lax.axis_index("s") * 2 + lax.axis_index("c")
        b = wid // 2
        h = wid % 2

        pltpu.sync_copy(keep_hbm.at[b], keep_v)
        pltpu.sync_copy(lenin_hbm, lenin_v)
        lane0 = lax.iota(jnp.int32, _L)
        lb = jnp.sum(jnp.where(lane0 == b, lenin_v[...], 0))

        def zrow(r, carry):
            for kk in range(NV_F):
                zbuf[r, pl.ds(kk * _L, _L)] = jnp.zeros((_L,), jnp.float32)
            return carry
        lax.fori_loop(0, C, zrow, 0)

        base_row = b * T

        def cbody(i, off):
            lane = lax.iota(jnp.int32, _L)
            tloc = lane + i * _L
            m = jnp.where(tloc < lb, keep_v[pl.ds(i * _L, _L)], 0)
            key = lane + (1 - m) * _L
            _, sv = plsc.sort_key_val(key, tloc + base_row)
            idx_v[pl.ds(off, _L)] = sv
            return off + plsc.all_reduce_population_count(m > 0)[0]

        cnt = lax.fori_loop(0, NV_T, cbody, jnp.int32(0))

        @pl.when(h == 0)
        def _():
            len_v[...] = jnp.zeros((_L,), jnp.int32) + cnt
            pltpu.sync_copy(len_v, len_hbm.at[b])

        v = jnp.clip(cnt - h * HALF, 0, HALF) * 0 + HALF

        def gstart(c, p):
            for kk in range(C // _L):
                src = idx_v[pl.ds(h * HALF + c * C + kk * _L, _L)]
                cidx[p][pl.ds(kk * _L, _L)] = jnp.clip(src, 0, BT - 1)
            pltpu.make_async_copy(tensor_hbm.at[cidx[p]], gbufs[p],
                                  gsem[p]).start()

        def gwait(p):
            pltpu.make_async_copy(tensor_hbm.at[cidx[p]], gbufs[p],
                                  gsem[p]).wait()

        def odesc(c, p, src=None):
            obase = base_row + h * HALF + c * C
            return pltpu.make_async_copy(
                gbufs[p] if src is None else src,
                out_hbm.at[pl.ds(obase, C)], osem[p])

        @pl.when(0 < v)
        def _():
            gstart(0, 0)

        def pair(c2, carry):
            for p in (0, 1):
                c = c2 * 2 + p
                q = 1 - p

                @pl.when(c >= 1)
                def _():
                    odesc(jnp.maximum(c - 1, 0), q).wait()

                @pl.when(((c + 1) < NCH) & ((c + 1) * C < v))
                def _():
                    gstart(c + 1, q)

                @pl.when(c * C < v)
                def _():
                    gwait(p)

                    @pl.when((c + 1) * C > v)
                    def _():
                        def mrow(r, carry2):
                            for kk in range(NV_F):
                                gbufs[p][r, pl.ds(kk * _L, _L)] = (
                                    jnp.zeros((_L,), jnp.float32))
                            return carry2
                        lax.fori_loop(jnp.maximum(v - c * C, 0), C, mrow, 0)

                    odesc(c, p).start()

                @pl.when(c * C >= v)
                def _():
                    odesc(c, p, src=zbuf).start()

            return carry

        lax.fori_loop(0, NCH // 2, pair, 0)

        odesc(NCH - 1, (NCH - 1) % 2).wait()

    return k


def kernel(tensor, lengths):
    B, T, F = tensor.shape
    keep = jnp.asarray(_keep_const(B, T))
    k = _sc_event_dropout(B, T, F)
    events_flat, lenbuf = k(tensor.reshape(B * T, F), keep,
                            lengths.astype(jnp.int32))
    return events_flat.reshape(B, T, F), lenbuf[:, 0]


# speedup vs baseline: 7.9718x; 7.9718x over previous
import functools

import jax
import jax.numpy as jnp
import numpy as np
from jax import lax
from jax.experimental import pallas as pl
from jax.experimental.pallas import tpu as pltpu
from jax.experimental.pallas import tpu_sc as plsc

_DROP_PROB = 0.1
_L = 16


@functools.lru_cache(maxsize=None)
def _keep_const(B, T):
    with jax.ensure_compile_time_eval():
        u = jax.random.uniform(jax.random.key(42), (B, T))
        return np.asarray(u > _DROP_PROB).astype(np.int32)


@functools.lru_cache(maxsize=None)
def _sc_event_dropout(B, T, F):
    BT = B * T
    C = 32
    HALF = T // 2
    NCH = HALF // C
    NV_T = T // _L
    NV_F = F // _L

    mesh = plsc.VectorSubcoreMesh(core_axis_name="c", subcore_axis_name="s")

    @functools.partial(
        pl.kernel,
        out_type=(
            jax.ShapeDtypeStruct((BT, F), jnp.float32),
            jax.ShapeDtypeStruct((B, _L), jnp.int32),
        ),
        mesh=mesh,
        compiler_params=pltpu.CompilerParams(needs_layout_passes=False),
        scratch_types=[
            pltpu.VMEM((T + _L,), jnp.int32),
            pltpu.VMEM((T,), jnp.int32),
            pltpu.VMEM((_L,), jnp.int32),
            pltpu.VMEM((_L,), jnp.int32),
            pltpu.VMEM((C,), jnp.int32),
            pltpu.VMEM((C,), jnp.int32),
            pltpu.VMEM((C, F), jnp.float32),
            pltpu.VMEM((C, F), jnp.float32),
            pltpu.VMEM((C, F), jnp.float32),
            pltpu.SemaphoreType.DMA,
            pltpu.SemaphoreType.DMA,
            pltpu.SemaphoreType.DMA,
            pltpu.SemaphoreType.DMA,
        ],
    )
    def k(tensor_hbm, keep_hbm, lenin_hbm, out_hbm, len_hbm,
          idx_v, keep_v, len_v, lenin_v, cidx0, cidx1, gbuf0, gbuf1, zbuf,
          gsem0, gsem1, osem0, osem1):
        cidx = (cidx0, cidx1)
        gbufs = (gbuf0, gbuf1)
        gsem = (gsem0, gsem1)
        osem = (osem0, osem1)
        wid = lax.axis_index("s") * 2 + lax.axis_index("c")
        b = wid // 2
        h = wid % 2

        pltpu.sync_copy(keep_hbm.at[b], keep_v)
        pltpu.sync_copy(lenin_hbm, lenin_v)
        lane0 = lax.iota(jnp.int32, _L)
        lb = jnp.sum(jnp.where(lane0 == b, lenin_v[...], 0))

        def zrow(r, carry):
            for kk in range(NV_F):
                zbuf[r, pl.ds(kk * _L, _L)] = jnp.zeros((_L,), jnp.float32)
            return carry
        lax.fori_loop(0, C, zrow, 0)

        base_row = b * T

        def cbody(i, off):
            lane = lax.iota(jnp.int32, _L)
            tloc = lane + i * _L
            m = jnp.where(tloc < lb, keep_v[pl.ds(i * _L, _L)], 0)
            key = lane + (1 - m) * _L
            _, sv = plsc.sort_key_val(key, tloc + base_row)
            idx_v[pl.ds(off, _L)] = sv
            return off + plsc.all_reduce_population_count(m > 0)[0]

        cnt = lax.fori_loop(0, NV_T, cbody, jnp.int32(0))

        v = jnp.clip(cnt - h * HALF, 0, HALF)

        def gstart(c, p):
            for kk in range(C // _L):
                src = idx_v[pl.ds(h * HALF + c * C + kk * _L, _L)]
                cidx[p][pl.ds(kk * _L, _L)] = jnp.clip(src, 0, BT - 1)
            pltpu.make_async_copy(tensor_hbm.at[cidx[p]], gbufs[p],
                                  gsem[p]).start()

        def gwait(p):
            pltpu.make_async_copy(tensor_hbm.at[cidx[p]], gbufs[p],
                                  gsem[p]).wait()

        def odesc(c, p, src=None):
            obase = base_row + h * HALF + c * C
            return pltpu.make_async_copy(
                gbufs[p] if src is None else src,
                out_hbm.at[pl.ds(obase, C)], osem[p])

        @pl.when(0 < v)
        def _():
            gstart(0, 0)

        def pair(c2, carry):
            for p in (0, 1):
                c = c2 * 2 + p
                q = 1 - p

                @pl.when(c * C < v)
                def _():
                    gwait(p)

                    @pl.when((c + 1) * C > v)
                    def _():
                        def mrow(r, carry2):
                            for kk in range(NV_F):
                                gbufs[p][r, pl.ds(kk * _L, _L)] = (
                                    jnp.zeros((_L,), jnp.float32))
                            return carry2
                        lax.fori_loop(jnp.maximum(v - c * C, 0), C, mrow, 0)

                    odesc(c, p).start()

                @pl.when(c * C >= v)
                def _():
                    odesc(c, p, src=zbuf).start()

                @pl.when(c >= 1)
                def _():
                    odesc(jnp.maximum(c - 1, 0), q).wait()

                @pl.when(((c + 1) < NCH) & ((c + 1) * C < v))
                def _():
                    gstart(c + 1, q)

            return carry

        lax.fori_loop(0, NCH // 2, pair, 0)

        @pl.when(h == 0)
        def _():
            len_v[...] = jnp.zeros((_L,), jnp.int32) + cnt
            pltpu.sync_copy(len_v, len_hbm.at[b])

        odesc(NCH - 1, (NCH - 1) % 2).wait()

    return k


def kernel(tensor, lengths):
    B, T, F = tensor.shape
    keep = jnp.asarray(_keep_const(B, T))
    k = _sc_event_dropout(B, T, F)
    events_flat, lenbuf = k(tensor.reshape(B * T, F), keep,
                            lengths.astype(jnp.int32))
    return events_flat.reshape(B, T, F), lenbuf[:, 0]


# speedup vs baseline: 8.7245x; 1.0944x over previous
import functools

import jax
import jax.numpy as jnp
import numpy as np
from jax import lax
from jax.experimental import pallas as pl
from jax.experimental.pallas import tpu as pltpu
from jax.experimental.pallas import tpu_sc as plsc

_DROP_PROB = 0.1
_L = 16


@functools.lru_cache(maxsize=None)
def _keep_const(B, T):
    with jax.ensure_compile_time_eval():
        u = jax.random.uniform(jax.random.key(42), (B, T))
        return np.asarray(u > _DROP_PROB).astype(np.int32)


@functools.lru_cache(maxsize=None)
def _sc_event_dropout(B, T, F):
    BT = B * T
    C = 32
    HALF = T // 2
    NCH = HALF // C
    NV_T = T // _L
    NV_F = F // _L

    mesh = plsc.VectorSubcoreMesh(core_axis_name="c", subcore_axis_name="s")

    @functools.partial(
        pl.kernel,
        out_type=(
            jax.ShapeDtypeStruct((BT, F), jnp.float32),
            jax.ShapeDtypeStruct((B, _L), jnp.int32),
        ),
        mesh=mesh,
        compiler_params=pltpu.CompilerParams(needs_layout_passes=False),
        scratch_types=[
            pltpu.VMEM((T + _L,), jnp.int32),
            pltpu.VMEM((T,), jnp.int32),
            pltpu.VMEM((_L,), jnp.int32),
            pltpu.VMEM((_L,), jnp.int32),
            pltpu.VMEM((C,), jnp.int32),
            pltpu.VMEM((C,), jnp.int32),
            pltpu.VMEM((C,), jnp.int32),
            pltpu.VMEM((C, F), jnp.float32),
            pltpu.VMEM((C, F), jnp.float32),
            pltpu.VMEM((C, F), jnp.float32),
            pltpu.SemaphoreType.DMA,
            pltpu.SemaphoreType.DMA,
            pltpu.SemaphoreType.DMA,
            pltpu.SemaphoreType.DMA,
            pltpu.SemaphoreType.DMA,
            pltpu.SemaphoreType.DMA,
            pltpu.SemaphoreType.DMA,
        ],
    )
    def k(tensor_hbm, keep_hbm, lenin_hbm, out_hbm, len_hbm,
          idx_v, keep_v, len_v, lenin_v, cidx0, cidx1, cidx2,
          gbuf0, gbuf1, gbuf2, gsem0, gsem1, gsem2,
          osem0, osem1, osem2, zsem):
        cidx = (cidx0, cidx1, cidx2)
        gbufs = (gbuf0, gbuf1, gbuf2)
        gsem = (gsem0, gsem1, gsem2)
        osem = (osem0, osem1, osem2)
        wid = lax.axis_index("s") * 2 + lax.axis_index("c")
        b = wid // 2
        h = wid % 2

        pltpu.sync_copy(keep_hbm.at[b], keep_v)
        pltpu.sync_copy(lenin_hbm, lenin_v)
        lane0 = lax.iota(jnp.int32, _L)
        lb = jnp.sum(jnp.where(lane0 == b, lenin_v[...], 0))

        base_row = b * T

        def cbody(i, off):
            lane = lax.iota(jnp.int32, _L)
            tloc = lane + i * _L
            m = jnp.where(tloc < lb, keep_v[pl.ds(i * _L, _L)], 0)
            key = lane + (1 - m) * _L
            _, sv = plsc.sort_key_val(key, tloc + base_row)
            idx_v[pl.ds(off, _L)] = sv
            return off + plsc.all_reduce_population_count(m > 0)[0]

        cnt = lax.fori_loop(0, NV_T, cbody, jnp.int32(0))

        v = jnp.clip(cnt - h * HALF, 0, HALF)

        def gstart(c, p):
            for kk in range(C // _L):
                src = idx_v[pl.ds(h * HALF + c * C + kk * _L, _L)]
                cidx[p][pl.ds(kk * _L, _L)] = jnp.clip(src, 0, BT - 1)
            pltpu.make_async_copy(tensor_hbm.at[cidx[p]], gbufs[p],
                                  gsem[p]).start()

        def gwait(p):
            pltpu.make_async_copy(tensor_hbm.at[cidx[p]], gbufs[p],
                                  gsem[p]).wait()

        def odesc(c, p, src=None):
            obase = base_row + h * HALF + c * C
            return pltpu.make_async_copy(
                gbufs[p] if src is None else src,
                out_hbm.at[pl.ds(obase, C)], osem[p])

        ncv = (v + C - 1) // C

        @pl.when(ncv > 0)
        def _():
            gstart(0, 0)

        @pl.when(ncv > 1)
        def _():
            gstart(1, 1)

        def tri(c3, carry):
            for j in (0, 1, 2):
                c = c3 * 3 + j
                jn = (j + 2) % 3

                @pl.when((c >= 1) & (c < ncv))
                def _():
                    odesc(jnp.maximum(c - 1, 0), jn).wait()

                @pl.when(c + 2 < ncv)
                def _():
                    gstart(c + 2, jn)

                @pl.when(c < ncv)
                def _():
                    gwait(j)

                    @pl.when((c + 1) * C > v)
                    def _():
                        def mrow(r, carry2):
                            for kk in range(NV_F):
                                gbufs[j][r, pl.ds(kk * _L, _L)] = (
                                    jnp.zeros((_L,), jnp.float32))
                            return carry2
                        lax.fori_loop(jnp.maximum(v - c * C, 0), C, mrow, 0)

                    odesc(c, j).start()

            return carry

        lax.fori_loop(0, (NCH + 2) // 3, tri, 0)

        for j in (0, 1, 2):
            @pl.when((ncv > 0) & ((ncv - 1) % 3 == j))
            def _():
                odesc(jnp.maximum(ncv - 1, 0), j).wait()

        @pl.when(h == 0)
        def _():
            len_v[...] = jnp.zeros((_L,), jnp.int32) + cnt
            pltpu.sync_copy(len_v, len_hbm.at[b])

        def zdesc(c):
            obase = base_row + h * HALF + c * C
            return pltpu.make_async_copy(
                gbuf0, out_hbm.at[pl.ds(obase, C)], zsem)

        @pl.when(ncv < NCH)
        def _():
            def zrow(r, carry2):
                for kk in range(NV_F):
                    gbuf0[r, pl.ds(kk * _L, _L)] = jnp.zeros((_L,), jnp.float32)
                return carry2
            lax.fori_loop(0, C, zrow, 0)

        def zb(k_, carry):
            c = ncv + k_

            @pl.when((k_ >= 2) & (c < NCH))
            def _():
                zdesc(jnp.maximum(c - 2, 0)).wait()

            @pl.when(c < NCH)
            def _():
                zdesc(c).start()

            return carry

        lax.fori_loop(0, NCH, zb, 0)
        nz = NCH - ncv

        @pl.when(nz >= 1)
        def _():
            zdesc(NCH - 1).wait()

        @pl.when(nz >= 2)
        def _():
            zdesc(NCH - 2).wait()

    return k


def kernel(tensor, lengths):
    B, T, F = tensor.shape
    keep = jnp.asarray(_keep_const(B, T))
    k = _sc_event_dropout(B, T, F)
    events_flat, lenbuf = k(tensor.reshape(B * T, F), keep,
                            lengths.astype(jnp.int32))
    return events_flat.reshape(B, T, F), lenbuf[:, 0]
